# baseline (speedup 1.0000x reference)
import functools

import jax
import jax.numpy as jnp
from jax import lax
from jax.experimental import pallas as pl
from jax.experimental.pallas import tpu as pltpu
import jax.experimental.pallas.tpu_sc as plsc

N = 10000
E = 320000
D_IN = 128
D_HID = 128
D_OUT = 40

NC = 2
NS = 16
NW = NC * NS
EPT = E // NW
RPT = N // NS


def _make_sc_agg(D, K, B, R):
    CHT = EPT // K
    NB = CHT // B
    assert NB * K * B == EPT and (K * B) % 8 == 0
    assert R - 1 < B
    mesh = plsc.VectorSubcoreMesh(
        core_axis_name="c", subcore_axis_name="s", num_cores=NC, num_subcores=NS
    )

    @functools.partial(
        pl.kernel,
        out_type=jax.ShapeDtypeStruct((NC, N, D), jnp.float32),
        mesh=mesh,
        compiler_params=pltpu.CompilerParams(use_tc_tiling_on_sc=False),
        scratch_types=[
            pltpu.VMEM((2, B, K), jnp.int32),
            pltpu.VMEM((2, B, K), jnp.int32),
        ] + [pltpu.VMEM((K, D), jnp.float32) for _ in range(R)]
        + [pltpu.SemaphoreType.DMA for _ in range(R)]
        + [
            pltpu.SemaphoreType.DMA,
            pltpu.SemaphoreType.DMA,
            pltpu.SemaphoreType.DMA,
            pltpu.VMEM_SHARED((N, D), jnp.float32),
        ],
    )
    def agg(h_hbm, src_hbm, dst_hbm, out_hbm, src_v, dst_v, *rest):
        rows = rest[:R]
        gsems = rest[R:2 * R]
        sz, si0, si1, acc_sh = rest[2 * R:]
        rows0 = rows[0]
        c = lax.axis_index("c")
        s = lax.axis_index("s")
        wid = c * NS + s

        def zrow(i, carry):
            for j in range(D // 16):
                rows0[i, pl.ds(j * 16, 16)] = jnp.zeros((16,), jnp.float32)
            return carry

        lax.fori_loop(0, K, zrow, 0)
        q, r = divmod(RPT, K)
        zdescs = [
            pltpu.async_copy(rows0, acc_sh.at[pl.ds(s * RPT + k * K, K)], sz)
            for k in range(q)
        ]
        if r:
            zdescs.append(pltpu.async_copy(
                rows0.at[pl.ds(0, r)], acc_sh.at[pl.ds(s * RPT + q * K, r)], sz
            ))

        idx_descs = [None] * NB
        isems = (si0, si1)

        def fire_idx(b):
            par = b % 2
            idx_descs[b] = (
                pltpu.async_copy(
                    src_hbm.at[wid, pl.ds(b * B, B)], src_v.at[par], isems[par]
                ),
                pltpu.async_copy(
                    dst_hbm.at[wid, pl.ds(b * B, B)], dst_v.at[par], isems[par]
                ),
            )

        fire_idx(0)
        for d in zdescs:
            d.wait()
        plsc.subcore_barrier()

        gd = [None] * R

        def issue_gather(n):
            b, j = divmod(n, B)
            if j == 0:
                for d in idx_descs[b]:
                    d.wait()
            gd[n % R] = pltpu.async_copy(
                h_hbm.at[src_v.at[b % 2, j]], rows[n % R], gsems[n % R]
            )

        for n in range(min(R - 1, CHT)):
            issue_gather(n)
        for g in range(CHT):
            b, j = divmod(g, B)
            n = g + R - 1
            if n < CHT:
                issue_gather(n)
            gd[g % R].wait()
            pltpu.sync_copy(rows[g % R], acc_sh.at[dst_v.at[b % 2, j]], add=True)
            if j == 0 and b + 1 < NB:
                fire_idx(b + 1)

        plsc.subcore_barrier()
        pltpu.sync_copy(
            acc_sh.at[pl.ds(s * RPT, RPT)], out_hbm.at[c, pl.ds(s * RPT, RPT)]
        )

    return agg


_sc_agg = functools.cache(_make_sc_agg)

_GRID = 10
_BR = N // _GRID


def _row_spec(d):
    return pl.BlockSpec((_BR, d), lambda i: (i, 0))


def _agg_spec(d):
    return pl.BlockSpec((NC, _BR, d), lambda i: (0, i, 0))


def _full_spec(r, c):
    return pl.BlockSpec((r, c), lambda i: (0, 0))


def _tc_layer1(x_ref, agg_ref, w1r_ref, w1n_ref, b1_ref, h_ref, inv_ref):
    a = agg_ref[0] + agg_ref[1]
    inv = 1.0 / jnp.maximum(a[:, 128:129], 1.0)
    mean = a[:, :128] * inv
    h = (
        jnp.dot(x_ref[...], w1r_ref[...], preferred_element_type=jnp.float32)
        + jnp.dot(mean, w1n_ref[...], preferred_element_type=jnp.float32)
        + b1_ref[...]
    )
    h_ref[...] = jnp.maximum(h, 0.0)
    inv_ref[...] = inv


def _tc_layer2(
    h1_ref, agg_ref, inv_ref, w2r_ref, w2n_ref, b2_ref, w3r_ref, w3n_ref, b3_ref,
    r3_ref, n3_ref,
):
    mean = (agg_ref[0] + agg_ref[1]) * inv_ref[...]
    h2 = (
        jnp.dot(h1_ref[...], w2r_ref[...], preferred_element_type=jnp.float32)
        + jnp.dot(mean, w2n_ref[...], preferred_element_type=jnp.float32)
        + b2_ref[...]
    )
    h2 = jnp.maximum(h2, 0.0)
    r3_ref[...] = (
        jnp.dot(h2, w3r_ref[...], preferred_element_type=jnp.float32) + b3_ref[...]
    )
    n3_ref[...] = jnp.dot(h2, w3n_ref[...], preferred_element_type=jnp.float32)


def _tc_layer3(r3_ref, agg_ref, inv_ref, out_ref):
    logits = r3_ref[...] + (agg_ref[0] + agg_ref[1])[:, :D_OUT] * inv_ref[...]
    m = jnp.max(logits, axis=1, keepdims=True)
    lse = m + jnp.log(jnp.sum(jnp.exp(logits - m), axis=1, keepdims=True))
    out_ref[...] = logits - lse


def kernel(x, edge_index, W1r, W1n, b1, W2r, W2n, b2, W3r, W3n, b3):
    src = edge_index[0].astype(jnp.int32)
    dst = edge_index[1].astype(jnp.int32)
    src_a, dst_a = src.reshape(NW, 200, 50), dst.reshape(NW, 200, 50)
    src_b, dst_b = src.reshape(NW, 80, 125), dst.reshape(NW, 80, 125)

    x_pad = jnp.concatenate(
        [x, jnp.ones((N, 1), jnp.float32), jnp.zeros((N, 15), jnp.float32)], axis=1
    )
    agg1 = _sc_agg(144, 50, 20, 4)(x_pad, src_a, dst_a)

    h1, inv = pl.pallas_call(
        _tc_layer1,
        grid=(_GRID,),
        in_specs=[
            _row_spec(D_IN),
            _agg_spec(144),
            _full_spec(D_IN, D_HID),
            _full_spec(D_IN, D_HID),
            _full_spec(1, D_HID),
        ],
        out_specs=[_row_spec(D_HID), _row_spec(1)],
        out_shape=[
            jax.ShapeDtypeStruct((N, D_HID), jnp.float32),
            jax.ShapeDtypeStruct((N, 1), jnp.float32),
        ],
    )(x, agg1, W1r, W1n, b1.reshape(1, D_HID))

    agg2 = _sc_agg(128, 100, 10, 3)(h1, src.reshape(NW, 100, 100),
                                    dst.reshape(NW, 100, 100))

    W3n_pad = jnp.concatenate([W3n, jnp.zeros((D_HID, 8), jnp.float32)], axis=1)
    r3, n3 = pl.pallas_call(
        _tc_layer2,
        grid=(_GRID,),
        in_specs=[
            _row_spec(D_HID),
            _agg_spec(D_HID),
            _row_spec(1),
            _full_spec(D_HID, D_HID),
            _full_spec(D_HID, D_HID),
            _full_spec(1, D_HID),
            _full_spec(D_HID, D_OUT),
            _full_spec(D_HID, 48),
            _full_spec(1, D_OUT),
        ],
        out_specs=[_row_spec(D_OUT), _row_spec(48)],
        out_shape=[
            jax.ShapeDtypeStruct((N, D_OUT), jnp.float32),
            jax.ShapeDtypeStruct((N, 48), jnp.float32),
        ],
    )(h1, agg2, inv, W2r, W2n, b2.reshape(1, D_HID), W3r, W3n_pad,
      b3.reshape(1, D_OUT))

    agg3 = _sc_agg(48, 125, 8, 4)(n3, src_b, dst_b)

    out = pl.pallas_call(
        _tc_layer3,
        grid=(_GRID,),
        in_specs=[_row_spec(D_OUT), _agg_spec(48), _row_spec(1)],
        out_specs=_row_spec(D_OUT),
        out_shape=jax.ShapeDtypeStruct((N, D_OUT), jnp.float32),
    )(r3, agg3, inv)

    return out


# speedup vs baseline: 1.0187x; 1.0187x over previous
import functools

import jax
import jax.numpy as jnp
from jax import lax
from jax.experimental import pallas as pl
from jax.experimental.pallas import tpu as pltpu
import jax.experimental.pallas.tpu_sc as plsc

N = 10000
E = 320000
D_IN = 128
D_HID = 128
D_OUT = 40

NC = 2
NS = 16
NW = NC * NS
EPT = E // NW
RPT = N // NS


def _make_sc_agg(D, K, B, R):
    CHT = EPT // K
    NB = CHT // B
    assert NB * K * B == EPT and (K * B) % 8 == 0
    assert R - 1 < B
    mesh = plsc.VectorSubcoreMesh(
        core_axis_name="c", subcore_axis_name="s", num_cores=NC, num_subcores=NS
    )

    @functools.partial(
        pl.kernel,
        out_type=jax.ShapeDtypeStruct((NC, N, D), jnp.float32),
        mesh=mesh,
        compiler_params=pltpu.CompilerParams(use_tc_tiling_on_sc=False),
        scratch_types=[
            pltpu.VMEM((2, B, K), jnp.int32),
            pltpu.VMEM((2, B, K), jnp.int32),
        ] + [pltpu.VMEM((K, D), jnp.float32) for _ in range(R)]
        + [pltpu.SemaphoreType.DMA for _ in range(R)]
        + [
            pltpu.SemaphoreType.DMA,
            pltpu.SemaphoreType.DMA,
            pltpu.SemaphoreType.DMA,
            pltpu.VMEM_SHARED((N, D), jnp.float32),
        ],
    )
    def agg(h_hbm, src_hbm, dst_hbm, out_hbm, src_v, dst_v, *rest):
        rows = rest[:R]
        gsems = rest[R:2 * R]
        sz, si0, si1, acc_sh = rest[2 * R:]
        rows0 = rows[0]
        c = lax.axis_index("c")
        s = lax.axis_index("s")
        wid = c * NS + s

        def zrow(i, carry):
            for j in range(D // 16):
                rows0[i, pl.ds(j * 16, 16)] = jnp.zeros((16,), jnp.float32)
            return carry

        lax.fori_loop(0, K, zrow, 0)
        q, r = divmod(RPT, K)
        zdescs = [
            pltpu.async_copy(rows0, acc_sh.at[pl.ds(s * RPT + k * K, K)], sz)
            for k in range(q)
        ]
        if r:
            zdescs.append(pltpu.async_copy(
                rows0.at[pl.ds(0, r)], acc_sh.at[pl.ds(s * RPT + q * K, r)], sz
            ))

        idx_descs = [None] * NB
        isems = (si0, si1)

        def fire_idx(b):
            par = b % 2
            idx_descs[b] = (
                pltpu.async_copy(
                    src_hbm.at[wid, pl.ds(b * B, B)], src_v.at[par], isems[par]
                ),
                pltpu.async_copy(
                    dst_hbm.at[wid, pl.ds(b * B, B)], dst_v.at[par], isems[par]
                ),
            )

        fire_idx(0)
        for d in zdescs:
            d.wait()
        plsc.subcore_barrier()

        gd = [None] * R

        def issue_gather(n):
            b, j = divmod(n, B)
            if j == 0:
                for d in idx_descs[b]:
                    d.wait()
            gd[n % R] = pltpu.async_copy(
                h_hbm.at[src_v.at[b % 2, j]], rows[n % R], gsems[n % R]
            )

        for n in range(min(R - 1, CHT)):
            issue_gather(n)
        for g in range(CHT):
            b, j = divmod(g, B)
            n = g + R - 1
            if n < CHT:
                issue_gather(n)
            gd[g % R].wait()
            pltpu.sync_copy(rows[g % R], acc_sh.at[dst_v.at[b % 2, j]], add=True)
            if j == 0 and b + 1 < NB:
                fire_idx(b + 1)

        plsc.subcore_barrier()
        pltpu.sync_copy(
            acc_sh.at[pl.ds(s * RPT, RPT)], out_hbm.at[c, pl.ds(s * RPT, RPT)]
        )

    return agg


_sc_agg = functools.cache(_make_sc_agg)

_GRID = 10
_BR = N // _GRID


def _row_spec(d):
    return pl.BlockSpec((_BR, d), lambda i: (i, 0))


def _agg_spec(d):
    return pl.BlockSpec((NC, _BR, d), lambda i: (0, i, 0))


def _full_spec(r, c):
    return pl.BlockSpec((r, c), lambda i: (0, 0))


def _tc_layer1(x_ref, agg_ref, w1r_ref, w1n_ref, b1_ref, h_ref, inv_ref):
    a = agg_ref[0] + agg_ref[1]
    inv = 1.0 / jnp.maximum(a[:, 128:129], 1.0)
    mean = a[:, :128] * inv
    h = (
        jnp.dot(x_ref[...], w1r_ref[...], preferred_element_type=jnp.float32)
        + jnp.dot(mean, w1n_ref[...], preferred_element_type=jnp.float32)
        + b1_ref[...]
    )
    h_ref[...] = jnp.maximum(h, 0.0)
    inv_ref[...] = inv


def _tc_layer2(
    h1_ref, agg_ref, inv_ref, w2r_ref, w2n_ref, b2_ref, w3r_ref, w3n_ref, b3_ref,
    r3_ref, n3_ref,
):
    mean = (agg_ref[0] + agg_ref[1]) * inv_ref[...]
    h2 = (
        jnp.dot(h1_ref[...], w2r_ref[...], preferred_element_type=jnp.float32)
        + jnp.dot(mean, w2n_ref[...], preferred_element_type=jnp.float32)
        + b2_ref[...]
    )
    h2 = jnp.maximum(h2, 0.0)
    r3_ref[...] = (
        jnp.dot(h2, w3r_ref[...], preferred_element_type=jnp.float32) + b3_ref[...]
    )
    n3_ref[...] = jnp.dot(h2, w3n_ref[...], preferred_element_type=jnp.float32)


def _tc_layer3(r3_ref, agg_ref, inv_ref, out_ref):
    logits = r3_ref[...] + (agg_ref[0] + agg_ref[1])[:, :D_OUT] * inv_ref[...]
    m = jnp.max(logits, axis=1, keepdims=True)
    lse = m + jnp.log(jnp.sum(jnp.exp(logits - m), axis=1, keepdims=True))
    out_ref[...] = logits - lse


def kernel(x, edge_index, W1r, W1n, b1, W2r, W2n, b2, W3r, W3n, b3):
    src = edge_index[0].astype(jnp.int32)
    dst = edge_index[1].astype(jnp.int32)
    src_a, dst_a = src.reshape(NW, 200, 50), dst.reshape(NW, 200, 50)
    src_b, dst_b = src.reshape(NW, 80, 125), dst.reshape(NW, 80, 125)

    x_pad = jnp.concatenate(
        [x, jnp.ones((N, 1), jnp.float32), jnp.zeros((N, 15), jnp.float32)], axis=1
    )
    agg1 = _sc_agg(144, 50, 20, 4)(x_pad, src_a, dst_a)

    h1, inv = pl.pallas_call(
        _tc_layer1,
        grid=(_GRID,),
        in_specs=[
            _row_spec(D_IN),
            _agg_spec(144),
            _full_spec(D_IN, D_HID),
            _full_spec(D_IN, D_HID),
            _full_spec(1, D_HID),
        ],
        out_specs=[_row_spec(D_HID), _row_spec(1)],
        out_shape=[
            jax.ShapeDtypeStruct((N, D_HID), jnp.float32),
            jax.ShapeDtypeStruct((N, 1), jnp.float32),
        ],
    )(x, agg1, W1r, W1n, b1.reshape(1, D_HID))

    agg2 = _sc_agg(128, 50, 20, 5)(h1, src_a, dst_a)

    W3n_pad = jnp.concatenate([W3n, jnp.zeros((D_HID, 8), jnp.float32)], axis=1)
    r3, n3 = pl.pallas_call(
        _tc_layer2,
        grid=(_GRID,),
        in_specs=[
            _row_spec(D_HID),
            _agg_spec(D_HID),
            _row_spec(1),
            _full_spec(D_HID, D_HID),
            _full_spec(D_HID, D_HID),
            _full_spec(1, D_HID),
            _full_spec(D_HID, D_OUT),
            _full_spec(D_HID, 48),
            _full_spec(1, D_OUT),
        ],
        out_specs=[_row_spec(D_OUT), _row_spec(48)],
        out_shape=[
            jax.ShapeDtypeStruct((N, D_OUT), jnp.float32),
            jax.ShapeDtypeStruct((N, 48), jnp.float32),
        ],
    )(h1, agg2, inv, W2r, W2n, b2.reshape(1, D_HID), W3r, W3n_pad,
      b3.reshape(1, D_OUT))

    agg3 = _sc_agg(48, 125, 8, 6)(n3, src_b, dst_b)

    out = pl.pallas_call(
        _tc_layer3,
        grid=(_GRID,),
        in_specs=[_row_spec(D_OUT), _agg_spec(48), _row_spec(1)],
        out_specs=_row_spec(D_OUT),
        out_shape=jax.ShapeDtypeStruct((N, D_OUT), jnp.float32),
    )(r3, agg3, inv)

    return out


# speedup vs baseline: 1.1177x; 1.0972x over previous
import functools

import jax
import jax.numpy as jnp
from jax import lax
from jax.experimental import pallas as pl
from jax.experimental.pallas import tpu as pltpu
import jax.experimental.pallas.tpu_sc as plsc

N = 10000
E = 320000
D_IN = 128
D_HID = 128
D_OUT = 40

NC = 2
NS = 16
NW = NC * NS
EPT = E // NW
RPT = N // NS


def _make_sc_agg(D, K, B, R, with_deg=False):
    CHT = EPT // K
    NB = CHT // B
    assert NB * K * B == EPT and (K * B) % 8 == 0
    assert R - 1 < B
    mesh = plsc.VectorSubcoreMesh(
        core_axis_name="c", subcore_axis_name="s", num_cores=NC, num_subcores=NS
    )
    out_type = jax.ShapeDtypeStruct((NC, N, D), jnp.float32)
    deg_scratch = []
    if with_deg:
        out_type = [out_type, jax.ShapeDtypeStruct((NC, N, 16), jnp.float32)]
        deg_scratch = [
            pltpu.VMEM((K, 16), jnp.float32),
            pltpu.VMEM((K, 16), jnp.float32),
            pltpu.VMEM_SHARED((N, 16), jnp.float32),
        ]

    @functools.partial(
        pl.kernel,
        out_type=out_type,
        mesh=mesh,
        compiler_params=pltpu.CompilerParams(use_tc_tiling_on_sc=False),
        scratch_types=[
            pltpu.VMEM((2, B, K), jnp.int32),
            pltpu.VMEM((2, B, K), jnp.int32),
        ] + [pltpu.VMEM((K, D), jnp.float32) for _ in range(R)]
        + [pltpu.SemaphoreType.DMA for _ in range(R)]
        + [
            pltpu.SemaphoreType.DMA,
            pltpu.SemaphoreType.DMA,
            pltpu.SemaphoreType.DMA,
            pltpu.VMEM_SHARED((N, D), jnp.float32),
        ] + deg_scratch,
    )
    def agg(h_hbm, src_hbm, dst_hbm, *rest):
        if with_deg:
            out_hbm, deg_hbm = rest[0], rest[1]
            rest = rest[2:]
        else:
            out_hbm = rest[0]
            rest = rest[1:]
        src_v, dst_v = rest[0], rest[1]
        rows = rest[2:2 + R]
        gsems = rest[2 + R:2 + 2 * R]
        sz, si0, si1, acc_sh = rest[2 + 2 * R:2 + 2 * R + 4]
        if with_deg:
            ones_v, zdeg_v, deg_sh = rest[2 + 2 * R + 4:]
        rows0 = rows[0]
        c = lax.axis_index("c")
        s = lax.axis_index("s")
        wid = c * NS + s

        def zrow(i, carry):
            for j in range(D // 16):
                rows0[i, pl.ds(j * 16, 16)] = jnp.zeros((16,), jnp.float32)
            if with_deg:
                ones_v[i, pl.ds(0, 16)] = jnp.ones((16,), jnp.float32)
                zdeg_v[i, pl.ds(0, 16)] = jnp.zeros((16,), jnp.float32)
            return carry

        lax.fori_loop(0, K, zrow, 0)
        q, r = divmod(RPT, K)
        zdescs = [
            pltpu.async_copy(rows0, acc_sh.at[pl.ds(s * RPT + k * K, K)], sz)
            for k in range(q)
        ]
        if r:
            zdescs.append(pltpu.async_copy(
                rows0.at[pl.ds(0, r)], acc_sh.at[pl.ds(s * RPT + q * K, r)], sz
            ))
        if with_deg:
            zdescs += [
                pltpu.async_copy(zdeg_v, deg_sh.at[pl.ds(s * RPT + k * K, K)], sz)
                for k in range(q)
            ]
            if r:
                zdescs.append(pltpu.async_copy(
                    zdeg_v.at[pl.ds(0, r)],
                    deg_sh.at[pl.ds(s * RPT + q * K, r)], sz
                ))

        idx_descs = [None] * NB
        isems = (si0, si1)

        def fire_idx(b):
            par = b % 2
            idx_descs[b] = (
                pltpu.async_copy(
                    src_hbm.at[wid, pl.ds(b * B, B)], src_v.at[par], isems[par]
                ),
                pltpu.async_copy(
                    dst_hbm.at[wid, pl.ds(b * B, B)], dst_v.at[par], isems[par]
                ),
            )

        fire_idx(0)
        for d in zdescs:
            d.wait()
        plsc.subcore_barrier()

        gd = [None] * R

        def issue_gather(n):
            b, j = divmod(n, B)
            if j == 0:
                for d in idx_descs[b]:
                    d.wait()
            gd[n % R] = pltpu.async_copy(
                h_hbm.at[src_v.at[b % 2, j]], rows[n % R], gsems[n % R]
            )

        for n in range(min(R - 1, CHT)):
            issue_gather(n)
        for g in range(CHT):
            b, j = divmod(g, B)
            n = g + R - 1
            if n < CHT:
                issue_gather(n)
            gd[g % R].wait()
            pltpu.sync_copy(rows[g % R], acc_sh.at[dst_v.at[b % 2, j]], add=True)
            if with_deg:
                pltpu.sync_copy(ones_v, deg_sh.at[dst_v.at[b % 2, j]], add=True)
            if j == 0 and b + 1 < NB:
                fire_idx(b + 1)

        plsc.subcore_barrier()
        pltpu.sync_copy(
            acc_sh.at[pl.ds(s * RPT, RPT)], out_hbm.at[c, pl.ds(s * RPT, RPT)]
        )
        if with_deg:
            pltpu.sync_copy(
                deg_sh.at[pl.ds(s * RPT, RPT)], deg_hbm.at[c, pl.ds(s * RPT, RPT)]
            )

    return agg


_sc_agg = functools.cache(_make_sc_agg)

_GRID = 10
_BR = N // _GRID


def _row_spec(d):
    return pl.BlockSpec((_BR, d), lambda i: (i, 0))


def _agg_spec(d):
    return pl.BlockSpec((NC, _BR, d), lambda i: (0, i, 0))


def _full_spec(r, c):
    return pl.BlockSpec((r, c), lambda i: (0, 0))


def _tc_layer1(x_ref, agg_ref, deg_ref, w1r_ref, w1n_ref, b1_ref, h_ref, inv_ref):
    deg = deg_ref[0, :, :1] + deg_ref[1, :, :1]
    inv = 1.0 / jnp.maximum(deg, 1.0)
    mean = (agg_ref[0] + agg_ref[1]) * inv
    h = (
        jnp.dot(x_ref[...], w1r_ref[...], preferred_element_type=jnp.float32)
        + jnp.dot(mean, w1n_ref[...], preferred_element_type=jnp.float32)
        + b1_ref[...]
    )
    h_ref[...] = jnp.maximum(h, 0.0)
    inv_ref[...] = inv


def _tc_layer2(
    h1_ref, agg_ref, inv_ref, w2r_ref, w2n_ref, b2_ref, w3r_ref, w3n_ref, b3_ref,
    r3_ref, n3_ref,
):
    mean = (agg_ref[0] + agg_ref[1]) * inv_ref[...]
    h2 = (
        jnp.dot(h1_ref[...], w2r_ref[...], preferred_element_type=jnp.float32)
        + jnp.dot(mean, w2n_ref[...], preferred_element_type=jnp.float32)
        + b2_ref[...]
    )
    h2 = jnp.maximum(h2, 0.0)
    r3_ref[...] = (
        jnp.dot(h2, w3r_ref[...], preferred_element_type=jnp.float32) + b3_ref[...]
    )
    n3_ref[...] = jnp.dot(h2, w3n_ref[...], preferred_element_type=jnp.float32)


def _tc_layer3(r3_ref, agg_ref, inv_ref, out_ref):
    logits = r3_ref[...] + (agg_ref[0] + agg_ref[1])[:, :D_OUT] * inv_ref[...]
    m = jnp.max(logits, axis=1, keepdims=True)
    lse = m + jnp.log(jnp.sum(jnp.exp(logits - m), axis=1, keepdims=True))
    out_ref[...] = logits - lse


def kernel(x, edge_index, W1r, W1n, b1, W2r, W2n, b2, W3r, W3n, b3):
    src = edge_index[0].astype(jnp.int32)
    dst = edge_index[1].astype(jnp.int32)
    src_a, dst_a = src.reshape(NW, 200, 50), dst.reshape(NW, 200, 50)
    src_b, dst_b = src.reshape(NW, 80, 125), dst.reshape(NW, 80, 125)

    agg1, degs = _sc_agg(128, 50, 20, 4, True)(x, src_a, dst_a)

    h1, inv = pl.pallas_call(
        _tc_layer1,
        grid=(_GRID,),
        in_specs=[
            _row_spec(D_IN),
            _agg_spec(D_IN),
            _agg_spec(16),
            _full_spec(D_IN, D_HID),
            _full_spec(D_IN, D_HID),
            _full_spec(1, D_HID),
        ],
        out_specs=[_row_spec(D_HID), _row_spec(1)],
        out_shape=[
            jax.ShapeDtypeStruct((N, D_HID), jnp.float32),
            jax.ShapeDtypeStruct((N, 1), jnp.float32),
        ],
    )(x, agg1, degs, W1r, W1n, b1.reshape(1, D_HID))

    agg2 = _sc_agg(128, 50, 20, 5)(h1, src_a, dst_a)

    W3n_pad = jnp.concatenate([W3n, jnp.zeros((D_HID, 8), jnp.float32)], axis=1)
    r3, n3 = pl.pallas_call(
        _tc_layer2,
        grid=(_GRID,),
        in_specs=[
            _row_spec(D_HID),
            _agg_spec(D_HID),
            _row_spec(1),
            _full_spec(D_HID, D_HID),
            _full_spec(D_HID, D_HID),
            _full_spec(1, D_HID),
            _full_spec(D_HID, D_OUT),
            _full_spec(D_HID, 48),
            _full_spec(1, D_OUT),
        ],
        out_specs=[_row_spec(D_OUT), _row_spec(48)],
        out_shape=[
            jax.ShapeDtypeStruct((N, D_OUT), jnp.float32),
            jax.ShapeDtypeStruct((N, 48), jnp.float32),
        ],
    )(h1, agg2, inv, W2r, W2n, b2.reshape(1, D_HID), W3r, W3n_pad,
      b3.reshape(1, D_OUT))

    agg3 = _sc_agg(48, 125, 8, 6)(n3, src_b, dst_b)

    out = pl.pallas_call(
        _tc_layer3,
        grid=(_GRID,),
        in_specs=[_row_spec(D_OUT), _agg_spec(48), _row_spec(1)],
        out_specs=_row_spec(D_OUT),
        out_shape=jax.ShapeDtypeStruct((N, D_OUT), jnp.float32),
    )(r3, agg3, inv)

    return out


# speedup vs baseline: 1.1286x; 1.0097x over previous
import functools

import jax
import jax.numpy as jnp
from jax import lax
from jax.experimental import pallas as pl
from jax.experimental.pallas import tpu as pltpu
import jax.experimental.pallas.tpu_sc as plsc

N = 10000
E = 320000
D_IN = 128
D_HID = 128
D_OUT = 40

NC = 2
NS = 16
NW = NC * NS
EPT = E // NW
RPT = N // NS


def _make_sc_agg(D, K, B, R, with_deg=False):
    CHT = EPT // K
    NB = CHT // B
    assert NB * K * B == EPT and (K * B) % 8 == 0
    assert R - 1 < B
    mesh = plsc.VectorSubcoreMesh(
        core_axis_name="c", subcore_axis_name="s", num_cores=NC, num_subcores=NS
    )
    out_type = jax.ShapeDtypeStruct((NC, N, D), jnp.float32)
    deg_scratch = []
    if with_deg:
        out_type = [out_type, jax.ShapeDtypeStruct((NC, N, 16), jnp.float32)]
        deg_scratch = [
            pltpu.VMEM((K, 16), jnp.float32),
            pltpu.VMEM((K, 16), jnp.float32),
            pltpu.VMEM_SHARED((N, 16), jnp.float32),
        ]

    @functools.partial(
        pl.kernel,
        out_type=out_type,
        mesh=mesh,
        compiler_params=pltpu.CompilerParams(use_tc_tiling_on_sc=False),
        scratch_types=[
            pltpu.VMEM((2, B, K), jnp.int32),
            pltpu.VMEM((2, B, K), jnp.int32),
        ] + [pltpu.VMEM((K, D), jnp.float32) for _ in range(R)]
        + [pltpu.SemaphoreType.DMA for _ in range(2 * R)]
        + [
            pltpu.SemaphoreType.DMA,
            pltpu.SemaphoreType.DMA,
            pltpu.SemaphoreType.DMA,
            pltpu.VMEM_SHARED((N, D), jnp.float32),
        ] + deg_scratch,
    )
    def agg(h_hbm, src_hbm, dst_hbm, *rest):
        if with_deg:
            out_hbm, deg_hbm = rest[0], rest[1]
            rest = rest[2:]
        else:
            out_hbm = rest[0]
            rest = rest[1:]
        src_v, dst_v = rest[0], rest[1]
        rows = rest[2:2 + R]
        gsems = rest[2 + R:2 + 2 * R]
        ssems = rest[2 + 2 * R:2 + 3 * R]
        sz, si0, si1, acc_sh = rest[2 + 3 * R:2 + 3 * R + 4]
        if with_deg:
            ones_v, zdeg_v, deg_sh = rest[2 + 3 * R + 4:]
        rows0 = rows[0]
        c = lax.axis_index("c")
        s = lax.axis_index("s")
        wid = c * NS + s

        def zrow(i, carry):
            for j in range(D // 16):
                rows0[i, pl.ds(j * 16, 16)] = jnp.zeros((16,), jnp.float32)
            if with_deg:
                ones_v[i, pl.ds(0, 16)] = jnp.ones((16,), jnp.float32)
                zdeg_v[i, pl.ds(0, 16)] = jnp.zeros((16,), jnp.float32)
            return carry

        lax.fori_loop(0, K, zrow, 0)
        q, r = divmod(RPT, K)
        zdescs = [
            pltpu.async_copy(rows0, acc_sh.at[pl.ds(s * RPT + k * K, K)], sz)
            for k in range(q)
        ]
        if r:
            zdescs.append(pltpu.async_copy(
                rows0.at[pl.ds(0, r)], acc_sh.at[pl.ds(s * RPT + q * K, r)], sz
            ))
        if with_deg:
            zdescs += [
                pltpu.async_copy(zdeg_v, deg_sh.at[pl.ds(s * RPT + k * K, K)], sz)
                for k in range(q)
            ]
            if r:
                zdescs.append(pltpu.async_copy(
                    zdeg_v.at[pl.ds(0, r)],
                    deg_sh.at[pl.ds(s * RPT + q * K, r)], sz
                ))

        idx_descs = [None] * NB
        isems = (si0, si1)

        def fire_idx(b):
            par = b % 2
            idx_descs[b] = (
                pltpu.async_copy(
                    src_hbm.at[wid, pl.ds(b * B, B)], src_v.at[par], isems[par]
                ),
                pltpu.async_copy(
                    dst_hbm.at[wid, pl.ds(b * B, B)], dst_v.at[par], isems[par]
                ),
            )

        fire_idx(0)
        for d in zdescs:
            d.wait()
        plsc.subcore_barrier()

        gd = [None] * R
        sd = [None] * CHT

        def drain_scatter(i):
            if 0 <= i < CHT and sd[i] is not None:
                sd[i].wait()
                sd[i] = None

        def issue_gather(n):
            b, j = divmod(n, B)
            if j == 0:
                for d in idx_descs[b]:
                    d.wait()
            drain_scatter(n - R)
            gd[n % R] = pltpu.async_copy(
                h_hbm.at[src_v.at[b % 2, j]], rows[n % R], gsems[n % R]
            )

        for n in range(min(R - 1, CHT)):
            issue_gather(n)
        for g in range(CHT):
            b, j = divmod(g, B)
            n = g + R - 1
            if n < CHT:
                issue_gather(n)
            gd[g % R].wait()
            sd[g] = pltpu.async_copy(
                rows[g % R], acc_sh.at[dst_v.at[b % 2, j]], ssems[g % R], add=True
            )
            if with_deg:
                pltpu.sync_copy(ones_v, deg_sh.at[dst_v.at[b % 2, j]], add=True)
            if j == 0 and b + 1 < NB:
                for i in range(g - R + 1, g):
                    drain_scatter(i)
                fire_idx(b + 1)

        for i in range(CHT):
            drain_scatter(i)

        plsc.subcore_barrier()
        pltpu.sync_copy(
            acc_sh.at[pl.ds(s * RPT, RPT)], out_hbm.at[c, pl.ds(s * RPT, RPT)]
        )
        if with_deg:
            pltpu.sync_copy(
                deg_sh.at[pl.ds(s * RPT, RPT)], deg_hbm.at[c, pl.ds(s * RPT, RPT)]
            )

    return agg


_sc_agg = functools.cache(_make_sc_agg)

_GRID = 10
_BR = N // _GRID


def _row_spec(d):
    return pl.BlockSpec((_BR, d), lambda i: (i, 0))


def _agg_spec(d):
    return pl.BlockSpec((NC, _BR, d), lambda i: (0, i, 0))


def _full_spec(r, c):
    return pl.BlockSpec((r, c), lambda i: (0, 0))


def _tc_layer1(x_ref, agg_ref, deg_ref, w1r_ref, w1n_ref, b1_ref, h_ref, inv_ref):
    deg = deg_ref[0, :, :1] + deg_ref[1, :, :1]
    inv = 1.0 / jnp.maximum(deg, 1.0)
    mean = (agg_ref[0] + agg_ref[1]) * inv
    h = (
        jnp.dot(x_ref[...], w1r_ref[...], preferred_element_type=jnp.float32)
        + jnp.dot(mean, w1n_ref[...], preferred_element_type=jnp.float32)
        + b1_ref[...]
    )
    h_ref[...] = jnp.maximum(h, 0.0)
    inv_ref[...] = inv


def _tc_layer2(
    h1_ref, agg_ref, inv_ref, w2r_ref, w2n_ref, b2_ref, w3r_ref, w3n_ref, b3_ref,
    r3_ref, n3_ref,
):
    mean = (agg_ref[0] + agg_ref[1]) * inv_ref[...]
    h2 = (
        jnp.dot(h1_ref[...], w2r_ref[...], preferred_element_type=jnp.float32)
        + jnp.dot(mean, w2n_ref[...], preferred_element_type=jnp.float32)
        + b2_ref[...]
    )
    h2 = jnp.maximum(h2, 0.0)
    r3_ref[...] = (
        jnp.dot(h2, w3r_ref[...], preferred_element_type=jnp.float32) + b3_ref[...]
    )
    n3_ref[...] = jnp.dot(h2, w3n_ref[...], preferred_element_type=jnp.float32)


def _tc_layer3(r3_ref, agg_ref, inv_ref, out_ref):
    logits = r3_ref[...] + (agg_ref[0] + agg_ref[1])[:, :D_OUT] * inv_ref[...]
    m = jnp.max(logits, axis=1, keepdims=True)
    lse = m + jnp.log(jnp.sum(jnp.exp(logits - m), axis=1, keepdims=True))
    out_ref[...] = logits - lse


def kernel(x, edge_index, W1r, W1n, b1, W2r, W2n, b2, W3r, W3n, b3):
    src = edge_index[0].astype(jnp.int32)
    dst = edge_index[1].astype(jnp.int32)
    src_a, dst_a = src.reshape(NW, 200, 50), dst.reshape(NW, 200, 50)
    src_b, dst_b = src.reshape(NW, 80, 125), dst.reshape(NW, 80, 125)

    agg1, degs = _sc_agg(128, 50, 20, 4, True)(x, src_a, dst_a)

    h1, inv = pl.pallas_call(
        _tc_layer1,
        grid=(_GRID,),
        in_specs=[
            _row_spec(D_IN),
            _agg_spec(D_IN),
            _agg_spec(16),
            _full_spec(D_IN, D_HID),
            _full_spec(D_IN, D_HID),
            _full_spec(1, D_HID),
        ],
        out_specs=[_row_spec(D_HID), _row_spec(1)],
        out_shape=[
            jax.ShapeDtypeStruct((N, D_HID), jnp.float32),
            jax.ShapeDtypeStruct((N, 1), jnp.float32),
        ],
    )(x, agg1, degs, W1r, W1n, b1.reshape(1, D_HID))

    agg2 = _sc_agg(128, 50, 20, 5)(h1, src_a, dst_a)

    W3n_pad = jnp.concatenate([W3n, jnp.zeros((D_HID, 8), jnp.float32)], axis=1)
    r3, n3 = pl.pallas_call(
        _tc_layer2,
        grid=(_GRID,),
        in_specs=[
            _row_spec(D_HID),
            _agg_spec(D_HID),
            _row_spec(1),
            _full_spec(D_HID, D_HID),
            _full_spec(D_HID, D_HID),
            _full_spec(1, D_HID),
            _full_spec(D_HID, D_OUT),
            _full_spec(D_HID, 48),
            _full_spec(1, D_OUT),
        ],
        out_specs=[_row_spec(D_OUT), _row_spec(48)],
        out_shape=[
            jax.ShapeDtypeStruct((N, D_OUT), jnp.float32),
            jax.ShapeDtypeStruct((N, 48), jnp.float32),
        ],
    )(h1, agg2, inv, W2r, W2n, b2.reshape(1, D_HID), W3r, W3n_pad,
      b3.reshape(1, D_OUT))

    agg3 = _sc_agg(48, 125, 8, 6)(n3, src_b, dst_b)

    out = pl.pallas_call(
        _tc_layer3,
        grid=(_GRID,),
        in_specs=[_row_spec(D_OUT), _agg_spec(48), _row_spec(1)],
        out_specs=_row_spec(D_OUT),
        out_shape=jax.ShapeDtypeStruct((N, D_OUT), jnp.float32),
    )(r3, agg3, inv)

    return out
